# baseline (speedup 1.0000x reference)
import functools
import jax
import jax.numpy as jnp
from jax import lax
from jax.experimental import pallas as pl
from jax.experimental.pallas import tpu as pltpu
from jax.experimental.pallas import tpu_sc as plsc

N = 4096
A_TAIL = 1536
A_HEAD = N - A_TAIL

NC = 2
NS = 16
NW = NC * NS
AT_W = N // NW
ATAIL_W = A_TAIL // NW
RB = 8
CHUNKS = N // 16
UNROLL = 4


def _compute_block(buf, v_ref, red_v, out_v, blk):
    zero = jnp.zeros((16,), jnp.float32)

    def jbody(j, accs):
        accs = list(accs)
        for u in range(UNROLL):
            sl = pl.ds((j * UNROLL + u) * 16, 16)
            vc = v_ref[sl]
            for r in range(RB):
                accs[r] = accs[r] + buf[r, sl] * vc
        return tuple(accs)

    accs = lax.fori_loop(0, CHUNKS // UNROLL, jbody, (zero,) * RB)
    for r in range(RB):
        red_v[r, :] = accs[r]
    lanes = lax.iota(jnp.int32, 16)
    row_idx = lanes & (RB - 1)
    hsum = zero
    for j in range(16):
        hsum = hsum + plsc.load_gather(
            red_v, [row_idx, jnp.full((16,), j, jnp.int32)])
    plsc.store_scatter(out_v, [blk * RB + row_idx], hsum, mask=lanes < RB)


@functools.cache
def _get_sc_matvecs():
  mesh = plsc.VectorSubcoreMesh(core_axis_name="c", subcore_axis_name="s",
                                num_cores=NC, num_subcores=NS)

  @functools.partial(
      pl.kernel,
      out_type=(jax.ShapeDtypeStruct((N,), jnp.float32),
                jax.ShapeDtypeStruct((A_TAIL,), jnp.float32)),
      mesh=mesh,
      compiler_params=pltpu.CompilerParams(needs_layout_passes=False),
      scratch_types=[
          pltpu.VMEM((N,), jnp.float32),
          pltpu.VMEM((N,), jnp.float32),
          pltpu.VMEM((N,), jnp.float32),
          pltpu.VMEM((RB, N), jnp.float32),
          pltpu.VMEM((RB, N), jnp.float32),
          pltpu.VMEM((AT_W,), jnp.float32),
          pltpu.VMEM((ATAIL_W,), jnp.float32),
          pltpu.VMEM((RB, 16), jnp.float32),
          pltpu.VMEM((16,), jnp.float32),
          pltpu.SemaphoreType.DMA,
          pltpu.SemaphoreType.DMA,
      ],
  )
  def _sc_matvecs(AT_hbm, A_hbm, x_hbm, y_hbm, vs_hbm, cs_hbm, cons_hbm,
                aty_hbm, axt_hbm,
                xun_v, yun_v, tmp_v, buf0, buf1, atyv, axtv, red_v, cons_v,
                sem0, sem1):
    c = lax.axis_index("c")
    s = lax.axis_index("s")
    wid = s * NC + c

    pltpu.sync_copy(cons_hbm, cons_v)
    csv = cons_v[...]

    pltpu.sync_copy(x_hbm, xun_v)
    pltpu.sync_copy(vs_hbm, tmp_v)

    def unscale_x(j, _):
        sl = pl.ds(j * 16, 16)
        xun_v[sl] = xun_v[sl] / tmp_v[sl] * csv
        return 0

    lax.fori_loop(0, CHUNKS, unscale_x, 0)

    pltpu.sync_copy(y_hbm, yun_v)
    pltpu.sync_copy(cs_hbm, tmp_v)

    def unscale_y(j, _):
        sl = pl.ds(j * 16, 16)
        yun_v[sl] = yun_v[sl] / tmp_v[sl] * csv
        return 0

    lax.fori_loop(0, CHUNKS, unscale_y, 0)

    def do_matvec(M_hbm, row0, nrows, v_ref, out_v):
        nblk = nrows // RB
        pltpu.async_copy(M_hbm.at[pl.ds(row0, RB)], buf0, sem0)

        def outer(k, _):
            blk0 = k * 2
            pltpu.async_copy(
                M_hbm.at[pl.ds(row0 + (blk0 + 1) * RB, RB)], buf1, sem1)
            pltpu.make_async_copy(
                M_hbm.at[pl.ds(row0, RB)], buf0, sem0).wait()
            _compute_block(buf0, v_ref, red_v, out_v, blk0)

            @pl.when(blk0 + 2 < nblk)
            def _():
                pltpu.async_copy(
                    M_hbm.at[pl.ds(row0 + (blk0 + 2) * RB, RB)], buf0, sem0)

            pltpu.make_async_copy(
                M_hbm.at[pl.ds(row0, RB)], buf1, sem1).wait()
            _compute_block(buf1, v_ref, red_v, out_v, blk0 + 1)
            return 0

        lax.fori_loop(0, nblk // 2, outer, 0)

    do_matvec(AT_hbm, wid * AT_W, AT_W, yun_v, atyv)
    do_matvec(A_hbm, A_HEAD + wid * ATAIL_W, ATAIL_W, xun_v, axtv)

    pltpu.sync_copy(atyv, aty_hbm.at[pl.ds(wid * AT_W, AT_W)])
    pltpu.sync_copy(axtv, axt_hbm.at[pl.ds(wid * ATAIL_W, ATAIL_W)])

  return _sc_matvecs


BLK = 256
ABLK = A_HEAD // (N // BLK)
GRID = N // BLK

_VAR, _CVH, _AXH, _B, _QX, _C, _QUAD, _LIN, _VIOT = range(9)


def _tc1_kernel(cons_ref,
                Q_ref, A_ref, bh_ref, Iyh_ref,
                b_ref, c_ref, x_ref, y_ref,
                il_ref, iu_ref, l_ref, u_ref,
                vscale_ref, cscale_ref,
                qx_out, part_ref, acc_ref):
    i = pl.program_id(0)
    relu = jax.nn.relu
    cs = cons_ref[0]

    xun = x_ref[...] / vscale_ref[...] * cs
    yun = y_ref[...] / cscale_ref[...] * cs

    Qx = jnp.dot(Q_ref[...], xun, preferred_element_type=jnp.float32)
    Axh = jnp.dot(A_ref[...], xun, preferred_element_type=jnp.float32)
    qx_out[...] = Qx

    sl = pl.ds(i * BLK, BLK)
    xb = x_ref[sl, :] / vscale_ref[sl, :] * cs
    yb = y_ref[sl, :] / cscale_ref[sl, :] * cs
    b = b_ref[...]
    c = c_ref[...]

    var_vio = relu(l_ref[...] - xb) * il_ref[...] + \
        relu(xb - u_ref[...]) * iu_ref[...]
    cons_vio = bh_ref[...] - Axh
    cons_vio = cons_vio + relu(-cons_vio) * Iyh_ref[...]

    p_var = jnp.max(jnp.abs(var_vio))
    p_cvh = jnp.max(jnp.abs(cons_vio))
    p_axh = jnp.max(jnp.abs(Axh))
    p_b = jnp.max(jnp.abs(b))
    p_qx = jnp.max(jnp.abs(Qx))
    p_c = jnp.max(jnp.abs(c))
    s_quad = jnp.sum(xb * Qx)
    s_lin = jnp.sum(c * xb)
    s_vio = jnp.sum(b * yb)

    @pl.when(i == 0)
    def _init():
        for k in range(9):
            acc_ref[k] = 0.0

    acc_ref[_VAR] = jnp.maximum(acc_ref[_VAR], p_var)
    acc_ref[_CVH] = jnp.maximum(acc_ref[_CVH], p_cvh)
    acc_ref[_AXH] = jnp.maximum(acc_ref[_AXH], p_axh)
    acc_ref[_B] = jnp.maximum(acc_ref[_B], p_b)
    acc_ref[_QX] = jnp.maximum(acc_ref[_QX], p_qx)
    acc_ref[_C] = jnp.maximum(acc_ref[_C], p_c)
    acc_ref[_QUAD] = acc_ref[_QUAD] + s_quad
    acc_ref[_LIN] = acc_ref[_LIN] + s_lin
    acc_ref[_VIOT] = acc_ref[_VIOT] + s_vio

    @pl.when(i == GRID - 1)
    def _fin():
        for k in range(9):
            part_ref[k] = acc_ref[k]


def _tc2_kernel(cons_ref, part_ref,
                qx_ref, aty_ref, axt_ref, bt_ref, Iyt_ref,
                c_ref, y_ref, Iy_ref, il_ref, iu_ref, l_ref, u_ref,
                cscale_ref, out_ref):
    relu = jax.nn.relu
    cs = cons_ref[0]
    yun = y_ref[...] / cscale_ref[...] * cs

    Qx = qx_ref[...]
    ATy = aty_ref[...]
    c = c_ref[...]

    pg = c - ATy + Qx
    rpg = relu(pg)
    rng = relu(-pg)
    il = il_ref[...]
    iu = iu_ref[...]
    RCV = pg - rpg * il + rng * iu
    DR = relu(-yun) * Iy_ref[...]
    RC = rpg * il - rng * iu
    tm = jnp.where(RC > 0, l_ref[...], u_ref[...])

    cons_vio_t = bt_ref[...] - axt_ref[...]
    cons_vio_t = cons_vio_t + relu(-cons_vio_t) * Iyt_ref[...]

    m_var = part_ref[_VAR]
    m_cv = jnp.maximum(part_ref[_CVH], jnp.max(jnp.abs(cons_vio_t)))
    m_ax = jnp.maximum(part_ref[_AXH], jnp.max(jnp.abs(axt_ref[...])))
    t1 = jnp.maximum(m_var, m_cv) / \
        (1.0 + jnp.maximum(m_ax, part_ref[_B]))

    m_rcv = jnp.maximum(jnp.max(jnp.abs(RCV)), jnp.max(jnp.abs(DR)))
    m_aty = jnp.max(jnp.abs(ATy))
    t2 = m_rcv / (1.0 + jnp.maximum(part_ref[_QX],
                                    jnp.maximum(m_aty, part_ref[_C])))

    quad = part_ref[_QUAD]
    lin = part_ref[_LIN]
    vio = part_ref[_VIOT]
    rcc = jnp.sum(RC * tm)
    t3 = jnp.abs(quad + lin - vio - rcc) / (
        1.0 + jnp.maximum(jnp.abs(vio - 0.5 * quad),
                          jnp.abs(0.5 * quad + lin)))
    res = jnp.maximum(t1, jnp.maximum(t2, t3))
    out_ref[0] = res
    out_ref[1] = t1
    out_ref[2] = t2
    out_ref[3] = t3


def kernel(Q, A, AT, b, c, x, y, Iy, il, iu, l, u, vscale, cscale, cons_scale):
    xf = x.reshape(N)
    yf = y.reshape(N)
    vsf = vscale.reshape(N)
    csf = cscale.reshape(N)
    cons1 = cons_scale.reshape(1)
    cons16 = jnp.broadcast_to(cons1, (16,))

    aty, ax_tail = _get_sc_matvecs()(AT, A, xf, yf, vsf, csf, cons16)

    b2 = b.reshape(N, 1)
    c2 = c.reshape(N, 1)

    row_q = pl.BlockSpec((BLK, N), lambda i: (i, 0))
    row_a = pl.BlockSpec((ABLK, N), lambda i: (i, 0))
    vec_q = pl.BlockSpec((BLK, 1), lambda i: (i, 0))
    vec_a = pl.BlockSpec((ABLK, 1), lambda i: (i, 0))
    full_vec = pl.BlockSpec((N, 1), lambda i: (0, 0))

    qx, parts = pl.pallas_call(
        _tc1_kernel,
        grid=(GRID,),
        in_specs=[
            pl.BlockSpec(memory_space=pltpu.SMEM),
            row_q,
            row_a,
            vec_a,
            vec_a,
            vec_q,
            vec_q,
            full_vec,
            full_vec,
            vec_q,
            vec_q,
            vec_q,
            vec_q,
            full_vec,
            full_vec,
        ],
        out_specs=(vec_q, pl.BlockSpec(memory_space=pltpu.SMEM)),
        out_shape=(jax.ShapeDtypeStruct((N, 1), jnp.float32),
                   jax.ShapeDtypeStruct((9,), jnp.float32)),
        scratch_shapes=[pltpu.SMEM((9,), jnp.float32)],
    )(cons1, Q, A, b2, Iy, b2, c2, x, y, il, iu, l, u, vscale, cscale)

    sq = (32, 128)
    st = (12, 128)
    out = pl.pallas_call(
        _tc2_kernel,
        in_specs=[pl.BlockSpec(memory_space=pltpu.SMEM)] * 2
        + [pl.BlockSpec(sq, lambda: (0, 0))] * 2
        + [pl.BlockSpec(st, lambda: (0, 0))] * 3
        + [pl.BlockSpec(sq, lambda: (0, 0))] * 8,
        out_specs=pl.BlockSpec(memory_space=pltpu.SMEM),
        out_shape=jax.ShapeDtypeStruct((4,), jnp.float32),
    )(cons1, parts,
      qx.reshape(sq), aty.reshape(sq),
      ax_tail.reshape(st), b[A_HEAD:].reshape(st), Iy[A_HEAD:].reshape(st),
      c.reshape(sq), y.reshape(sq), Iy.reshape(sq),
      il.reshape(sq), iu.reshape(sq), l.reshape(sq), u.reshape(sq),
      cscale.reshape(sq))

    res = out[0].reshape(1, 1)
    t1 = out[1].reshape(())
    t2 = out[2].reshape(())
    t3 = out[3].reshape(1, 1)
    return res, t1, t2, t3


# speedup vs baseline: 1.1644x; 1.1644x over previous
import functools
import jax
import jax.numpy as jnp
from jax import lax
from jax.experimental import pallas as pl
from jax.experimental.pallas import tpu as pltpu
from jax.experimental.pallas import tpu_sc as plsc

N = 4096
A_TAIL = 1536
A_HEAD = N - A_TAIL

NC = 2
NS = 16
NW = NC * NS
AT_W = N // NW
ATAIL_W = A_TAIL // NW
RB = 8
CHUNKS = N // 16
UNROLL = 4


def _compute_block(buf, v_ref, red_v, out_v, blk):
    zero = jnp.zeros((16,), jnp.float32)

    def jbody(j, accs):
        accs = list(accs)
        for u in range(UNROLL):
            sl = pl.ds((j * UNROLL + u) * 16, 16)
            vc = v_ref[sl]
            for r in range(RB):
                accs[r] = accs[r] + buf[r, sl] * vc
        return tuple(accs)

    accs = lax.fori_loop(0, CHUNKS // UNROLL, jbody, (zero,) * RB)
    for r in range(RB):
        red_v[r, :] = accs[r]
    lanes = lax.iota(jnp.int32, 16)
    row_idx = lanes & (RB - 1)
    hsum = zero
    for j in range(16):
        hsum = hsum + plsc.load_gather(
            red_v, [row_idx, jnp.full((16,), j, jnp.int32)])
    plsc.store_scatter(out_v, [blk * RB + row_idx], hsum, mask=lanes < RB)


@functools.cache
def _get_sc_matvecs():
  mesh = plsc.VectorSubcoreMesh(core_axis_name="c", subcore_axis_name="s",
                                num_cores=NC, num_subcores=NS)

  @functools.partial(
      pl.kernel,
      out_type=(jax.ShapeDtypeStruct((N,), jnp.float32),
                jax.ShapeDtypeStruct((A_TAIL,), jnp.float32)),
      mesh=mesh,
      compiler_params=pltpu.CompilerParams(needs_layout_passes=False),
      scratch_types=[
          pltpu.VMEM((N,), jnp.float32),
          pltpu.VMEM((N,), jnp.float32),
          pltpu.VMEM((N,), jnp.float32),
          pltpu.VMEM((RB, N), jnp.float32),
          pltpu.VMEM((RB, N), jnp.float32),
          pltpu.VMEM((AT_W,), jnp.float32),
          pltpu.VMEM((ATAIL_W,), jnp.float32),
          pltpu.VMEM((RB, 16), jnp.float32),
          pltpu.VMEM((16,), jnp.float32),
          pltpu.SemaphoreType.DMA,
          pltpu.SemaphoreType.DMA,
      ],
  )
  def _sc_matvecs(AT_hbm, A_hbm, x_hbm, y_hbm, vs_hbm, cs_hbm, cons_hbm,
                  aty_hbm, axt_hbm,
                  xun_v, yun_v, tmp_v, buf0, buf1, atyv, axtv, red_v, cons_v,
                  sem0, sem1):
    c = lax.axis_index("c")
    s = lax.axis_index("s")
    wid = s * NC + c

    pltpu.sync_copy(cons_hbm, cons_v)
    csv = cons_v[...]

    pltpu.sync_copy(x_hbm, xun_v)
    pltpu.sync_copy(vs_hbm, tmp_v)

    def unscale_x(j, _):
        sl = pl.ds(j * 16, 16)
        xun_v[sl] = xun_v[sl] / tmp_v[sl] * csv
        return 0

    lax.fori_loop(0, CHUNKS, unscale_x, 0)

    pltpu.sync_copy(y_hbm, yun_v)
    pltpu.sync_copy(cs_hbm, tmp_v)

    def unscale_y(j, _):
        sl = pl.ds(j * 16, 16)
        yun_v[sl] = yun_v[sl] / tmp_v[sl] * csv
        return 0

    lax.fori_loop(0, CHUNKS, unscale_y, 0)

    def do_matvec(M_hbm, row0, nrows, v_ref, out_v):
        nblk = nrows // RB
        pltpu.async_copy(M_hbm.at[pl.ds(row0, RB)], buf0, sem0)

        def outer(k, _):
            blk0 = k * 2
            pltpu.async_copy(
                M_hbm.at[pl.ds(row0 + (blk0 + 1) * RB, RB)], buf1, sem1)
            pltpu.make_async_copy(
                M_hbm.at[pl.ds(row0, RB)], buf0, sem0).wait()
            _compute_block(buf0, v_ref, red_v, out_v, blk0)

            @pl.when(blk0 + 2 < nblk)
            def _():
                pltpu.async_copy(
                    M_hbm.at[pl.ds(row0 + (blk0 + 2) * RB, RB)], buf0, sem0)

            pltpu.make_async_copy(
                M_hbm.at[pl.ds(row0, RB)], buf1, sem1).wait()
            _compute_block(buf1, v_ref, red_v, out_v, blk0 + 1)
            return 0

        lax.fori_loop(0, nblk // 2, outer, 0)

    do_matvec(AT_hbm, wid * AT_W, AT_W, yun_v, atyv)
    do_matvec(A_hbm, A_HEAD + wid * ATAIL_W, ATAIL_W, xun_v, axtv)

    pltpu.sync_copy(atyv, aty_hbm.at[pl.ds(wid * AT_W, AT_W)])
    pltpu.sync_copy(axtv, axt_hbm.at[pl.ds(wid * ATAIL_W, ATAIL_W)])

  return _sc_matvecs


BLK = 256
GRID = N // BLK
R2 = BLK // 128

_VAR, _B, _QX, _C, _QUAD, _LIN, _VIOT = range(7)


def _tcq_kernel(cons_ref, Q_ref, xc_ref, vsc_ref,
                x2_ref, vs2_ref, y2_ref, cs2_ref, b2_ref, c2_ref,
                il2_ref, iu2_ref, l2_ref, u2_ref,
                qx2_out, part_ref, acc_ref):
    i = pl.program_id(0)
    relu = jax.nn.relu
    cs = cons_ref[0]

    xun_c = xc_ref[...] / vsc_ref[...] * cs
    Qx2 = jnp.dot(Q_ref[...], xun_c,
                  preferred_element_type=jnp.float32).reshape(1, R2, 128)
    qx2_out[...] = Qx2

    xun2 = x2_ref[...] / vs2_ref[...] * cs
    yun2 = y2_ref[...] / cs2_ref[...] * cs
    b2 = b2_ref[...]
    c2 = c2_ref[...]

    var_vio = relu(l2_ref[...] - xun2) * il2_ref[...] + \
        relu(xun2 - u2_ref[...]) * iu2_ref[...]

    p_var = jnp.max(jnp.abs(var_vio))
    p_b = jnp.max(jnp.abs(b2))
    p_qx = jnp.max(jnp.abs(Qx2))
    p_c = jnp.max(jnp.abs(c2))
    s_quad = jnp.sum(xun2 * Qx2)
    s_lin = jnp.sum(c2 * xun2)
    s_vio = jnp.sum(b2 * yun2)

    @pl.when(i == 0)
    def _init():
        for k in range(7):
            acc_ref[k] = 0.0

    acc_ref[_VAR] = jnp.maximum(acc_ref[_VAR], p_var)
    acc_ref[_B] = jnp.maximum(acc_ref[_B], p_b)
    acc_ref[_QX] = jnp.maximum(acc_ref[_QX], p_qx)
    acc_ref[_C] = jnp.maximum(acc_ref[_C], p_c)
    acc_ref[_QUAD] = acc_ref[_QUAD] + s_quad
    acc_ref[_LIN] = acc_ref[_LIN] + s_lin
    acc_ref[_VIOT] = acc_ref[_VIOT] + s_vio

    @pl.when(i == GRID - 1)
    def _fin():
        for k in range(7):
            part_ref[k] = acc_ref[k]


AGRID = A_HEAD // BLK


def _tca_kernel(cons_ref, A_ref, xc_ref, vsc_ref, b2_ref, Iy2_ref,
                part_ref, acc_ref):
    i = pl.program_id(0)
    relu = jax.nn.relu
    cs = cons_ref[0]

    xun_c = xc_ref[...] / vsc_ref[...] * cs
    Ax2 = jnp.dot(A_ref[...], xun_c,
                  preferred_element_type=jnp.float32).reshape(1, R2, 128)
    cv = b2_ref[...] - Ax2
    cv = cv + relu(-cv) * Iy2_ref[...]

    p_cv = jnp.max(jnp.abs(cv))
    p_ax = jnp.max(jnp.abs(Ax2))

    @pl.when(i == 0)
    def _init():
        acc_ref[0] = 0.0
        acc_ref[1] = 0.0

    acc_ref[0] = jnp.maximum(acc_ref[0], p_cv)
    acc_ref[1] = jnp.maximum(acc_ref[1], p_ax)

    @pl.when(i == AGRID - 1)
    def _fin():
        part_ref[0] = acc_ref[0]
        part_ref[1] = acc_ref[1]


def _tc3_kernel(cons_ref, pq_ref, pa_ref,
                qx2_ref, aty2_ref, axt2_ref, bt2_ref, Iyt2_ref,
                c2_ref, y2_ref, cs2_ref, Iy2_ref,
                il2_ref, iu2_ref, l2_ref, u2_ref, out_ref):
    relu = jax.nn.relu
    cs = cons_ref[0]
    yun = y2_ref[...] / cs2_ref[...] * cs

    Qx = qx2_ref[...]
    ATy = aty2_ref[...]
    c = c2_ref[...]

    pg = c - ATy + Qx
    rpg = relu(pg)
    rng = relu(-pg)
    il = il2_ref[...]
    iu = iu2_ref[...]
    RCV = pg - rpg * il + rng * iu
    DR = relu(-yun) * Iy2_ref[...]
    RC = rpg * il - rng * iu
    tm = jnp.where(RC > 0, l2_ref[...], u2_ref[...])

    cv_t = bt2_ref[...] - axt2_ref[...]
    cv_t = cv_t + relu(-cv_t) * Iyt2_ref[...]

    m_var = pq_ref[_VAR]
    m_cv = jnp.maximum(pa_ref[0], jnp.max(jnp.abs(cv_t)))
    m_ax = jnp.maximum(pa_ref[1], jnp.max(jnp.abs(axt2_ref[...])))
    t1 = jnp.maximum(m_var, m_cv) / (1.0 + jnp.maximum(m_ax, pq_ref[_B]))

    m_rcv = jnp.maximum(jnp.max(jnp.abs(RCV)), jnp.max(jnp.abs(DR)))
    m_aty = jnp.max(jnp.abs(ATy))
    t2 = m_rcv / (1.0 + jnp.maximum(pq_ref[_QX],
                                    jnp.maximum(m_aty, pq_ref[_C])))

    quad = pq_ref[_QUAD]
    lin = pq_ref[_LIN]
    vio = pq_ref[_VIOT]
    rcc = jnp.sum(RC * tm)
    t3 = jnp.abs(quad + lin - vio - rcc) / (
        1.0 + jnp.maximum(jnp.abs(vio - 0.5 * quad),
                          jnp.abs(0.5 * quad + lin)))
    res = jnp.maximum(t1, jnp.maximum(t2, t3))
    out_ref[0] = res
    out_ref[1] = t1
    out_ref[2] = t2
    out_ref[3] = t3


def kernel(Q, A, AT, b, c, x, y, Iy, il, iu, l, u, vscale, cscale, cons_scale):
    xf = x.reshape(N)
    yf = y.reshape(N)
    vsf = vscale.reshape(N)
    csf = cscale.reshape(N)
    cons1 = cons_scale.reshape(1)
    cons16 = jnp.broadcast_to(cons1, (16,))

    aty, ax_tail = _get_sc_matvecs()(AT, A, xf, yf, vsf, csf, cons16)

    sq = (32, 128)
    s3 = (GRID, R2, 128)
    x2 = x.reshape(s3)
    vs2 = vscale.reshape(s3)
    y2 = y.reshape(s3)
    cs2 = cscale.reshape(s3)
    b2 = b.reshape(s3)
    c2 = c.reshape(s3)
    Iy2 = Iy.reshape(s3)
    il2 = il.reshape(s3)
    iu2 = iu.reshape(s3)
    l2 = l.reshape(s3)
    u2 = u.reshape(s3)

    row_q = pl.BlockSpec((BLK, N), lambda i: (i, 0))
    v2 = pl.BlockSpec((1, R2, 128), lambda i: (i, 0, 0))
    full_col = pl.BlockSpec((N, 1), lambda i: (0, 0))
    smem = pl.BlockSpec(memory_space=pltpu.SMEM)

    qx2, parts_q = pl.pallas_call(
        _tcq_kernel,
        grid=(GRID,),
        in_specs=[smem, row_q, full_col, full_col] + [v2] * 10,
        out_specs=(v2, smem),
        out_shape=(jax.ShapeDtypeStruct(s3, jnp.float32),
                   jax.ShapeDtypeStruct((7,), jnp.float32)),
        scratch_shapes=[pltpu.SMEM((7,), jnp.float32)],
    )(cons1, Q, x, vscale, x2, vs2, y2, cs2, b2, c2, il2, iu2, l2, u2)

    parts_a = pl.pallas_call(
        _tca_kernel,
        grid=(AGRID,),
        in_specs=[smem, row_q, full_col, full_col, v2, v2],
        out_specs=smem,
        out_shape=jax.ShapeDtypeStruct((2,), jnp.float32),
        scratch_shapes=[pltpu.SMEM((2,), jnp.float32)],
    )(cons1, A, x, vscale, b2, Iy2)

    st = (A_TAIL // 128, 128)
    g0 = lambda: (0, 0)
    out = pl.pallas_call(
        _tc3_kernel,
        in_specs=[smem] * 3
        + [pl.BlockSpec(sq, g0)] * 2
        + [pl.BlockSpec(st, g0)] * 3
        + [pl.BlockSpec(sq, g0)] * 8,
        out_specs=smem,
        out_shape=jax.ShapeDtypeStruct((4,), jnp.float32),
    )(cons1, parts_q, parts_a,
      qx2.reshape(sq), aty.reshape(sq),
      ax_tail.reshape(st), b.reshape(sq)[A_HEAD // 128:],
      Iy.reshape(sq)[A_HEAD // 128:],
      c.reshape(sq), y.reshape(sq), cscale.reshape(sq), Iy.reshape(sq),
      il.reshape(sq), iu.reshape(sq), l.reshape(sq), u.reshape(sq))

    res = out[0].reshape(1, 1)
    t1 = out[1].reshape(())
    t2 = out[2].reshape(())
    t3 = out[3].reshape(1, 1)
    return res, t1, t2, t3
